# Initial kernel scaffold; baseline (speedup 1.0000x reference)
import functools

import jax
import jax.numpy as jnp
from jax import lax
from jax.experimental import pallas as pl
from jax.experimental.pallas import tpu as pltpu
from jax.experimental.pallas import tpu_sc as plsc

_D = 64
_B = 16384
_NC = 2
_NS = 16
_NW = _NC * _NS
_L = 16
_K = 128
_PW = _B // _NW
_NCHUNK = _PW // _K


def _sqrt16(x):
    i = plsc.bitcast(x, jnp.int32)
    y = plsc.bitcast((i >> 1) + jnp.int32(0x1FBD1DF5), jnp.float32)
    y = 0.5 * (y + x / y)
    y = 0.5 * (y + x / y)
    y = 0.5 * (y + x / y)
    return y


def _body(i0_hbm, i1_hbm, i2_hbm, cls_hbm, rel_hbm, out_hbm,
          i0_v, i1_v, i2_v, c_v, r_v, d_v, o_v, sem):
    wid = lax.axis_index("s") * _NC + lax.axis_index("c")
    base = wid * _PW

    lanes = lax.iota(jnp.int32, _L)

    for chunk in range(_NCHUNK):
        cb = base + chunk * _K
        pltpu.sync_copy(i0_hbm.at[pl.ds(cb, _K)], i0_v)
        pltpu.sync_copy(i1_hbm.at[pl.ds(cb, _K)], i1_v)
        pltpu.sync_copy(i2_hbm.at[pl.ds(cb, _K)], i2_v)
        cp0 = pltpu.async_copy(cls_hbm.at[i0_v], c_v, sem)
        cp1 = pltpu.async_copy(rel_hbm.at[i1_v], r_v, sem)
        cp2 = pltpu.async_copy(cls_hbm.at[i2_v], d_v, sem)
        cp0.wait()
        cp1.wait()
        cp2.wait()

        def group_body(g, _):
            rows = lanes + g * _L

            def dim_body(j, accs):
                acc_dst, acc_c, acc_d = accs
                jlo = jnp.full((_L,), 0, jnp.int32) + j
                jhi = jlo + _D
                c1 = plsc.load_gather(c_v, [rows, jlo])
                c2 = plsc.load_gather(c_v, [rows, jhi])
                d1 = plsc.load_gather(d_v, [rows, jlo])
                d2 = plsc.load_gather(d_v, [rows, jhi])
                r1 = plsc.load_gather(r_v, [rows, jlo])
                r2 = plsc.load_gather(r_v, [rows, jhi])
                t = (jnp.abs(c1 - d1 - r1) + jnp.abs(c2) + jnp.abs(d2)
                     - jnp.abs(r2))
                t = jnp.maximum(t, 0.0)
                return (acc_dst + t * t, acc_c + c1 * c1, acc_d + d1 * d1)

            zero = jnp.zeros((_L,), jnp.float32)
            acc_dst, acc_c, acc_d = lax.fori_loop(
                0, _D, dim_body, (zero, zero, zero))
            res = (_sqrt16(acc_dst)
                   + jnp.abs(_sqrt16(acc_c) - 1.0)
                   + jnp.abs(_sqrt16(acc_d) - 1.0))
            o_v[pl.ds(chunk * _K + g * _L, _L)] = res
            return 0

        lax.fori_loop(0, _K // _L, group_body, 0)

    pltpu.sync_copy(o_v, out_hbm.at[pl.ds(base, _PW)])


_mesh = plsc.VectorSubcoreMesh(core_axis_name="c", subcore_axis_name="s")

_tb = functools.partial(
    pl.kernel,
    out_type=jax.ShapeDtypeStruct((_B,), jnp.float32),
    mesh=_mesh,
    scratch_types=[
        pltpu.VMEM((_K,), jnp.int32),
        pltpu.VMEM((_K,), jnp.int32),
        pltpu.VMEM((_K,), jnp.int32),
        pltpu.VMEM((_K, 2 * _D), jnp.float32),
        pltpu.VMEM((_K, 2 * _D), jnp.float32),
        pltpu.VMEM((_K, 2 * _D), jnp.float32),
        pltpu.VMEM((_PW,), jnp.float32),
        pltpu.SemaphoreType.DMA,
    ],
)(_body)


@jax.jit
def kernel(input, class_emb, rel_emb):
    idx = input.astype(jnp.int32)
    out = _tb(idx[:, 0], idx[:, 1], idx[:, 2], class_emb, rel_emb)
    return out.reshape(_B, 1)


# speedup vs baseline: 1.5041x; 1.5041x over previous
import functools

import jax
import jax.numpy as jnp
from jax import lax
from jax.experimental import pallas as pl
from jax.experimental.pallas import tpu as pltpu
from jax.experimental.pallas import tpu_sc as plsc

_D = 64
_B = 16384
_NC = 2
_NS = 16
_NW = _NC * _NS
_L = 16
_K = 128
_PW = _B // _NW
_NCHUNK = _PW // _K


def _sqrt16(x):
    i = plsc.bitcast(x, jnp.int32)
    y = plsc.bitcast((i >> 1) + jnp.int32(0x1FBD1DF5), jnp.float32)
    y = 0.5 * (y + x / y)
    y = 0.5 * (y + x / y)
    y = 0.5 * (y + x / y)
    return y


def _body(i0_hbm, i1_hbm, i2_hbm, cls_hbm, rel_hbm, out_hbm,
          i0_v, i1_v, i2_v, c_v, r_v, d_v, o_v, sem):
    wid = lax.axis_index("s") * _NC + lax.axis_index("c")
    base = wid * _PW

    lanes = lax.iota(jnp.int32, _L)

    for chunk in range(_NCHUNK):
        cb = base + chunk * _K
        pltpu.sync_copy(i0_hbm.at[pl.ds(cb, _K)], i0_v)
        pltpu.sync_copy(i1_hbm.at[pl.ds(cb, _K)], i1_v)
        pltpu.sync_copy(i2_hbm.at[pl.ds(cb, _K)], i2_v)
        cp0 = pltpu.async_copy(cls_hbm.at[i0_v], c_v, sem)
        cp1 = pltpu.async_copy(rel_hbm.at[i1_v], r_v, sem)
        cp2 = pltpu.async_copy(cls_hbm.at[i2_v], d_v, sem)
        cp0.wait()
        cp1.wait()
        cp2.wait()

        def group_body(g, _):
            rows = lanes + g * _L

            def dim_body(j, accs):
                acc_dst, acc_c, acc_d = accs
                jlo = jnp.full((_L,), 0, jnp.int32) + j
                jhi = jlo + _D
                c1 = plsc.load_gather(c_v, [rows, jlo])
                c2 = plsc.load_gather(c_v, [rows, jhi])
                d1 = plsc.load_gather(d_v, [rows, jlo])
                d2 = plsc.load_gather(d_v, [rows, jhi])
                r1 = plsc.load_gather(r_v, [rows, jlo])
                r2 = plsc.load_gather(r_v, [rows, jhi])
                t = (jnp.abs(c1 - d1 - r1) + jnp.abs(c2) + jnp.abs(d2)
                     - jnp.abs(r2))
                t = jnp.maximum(t, 0.0)
                return (acc_dst + t * t, acc_c + c1 * c1, acc_d + d1 * d1)

            zero = jnp.zeros((_L,), jnp.float32)
            acc_dst, acc_c, acc_d = lax.fori_loop(
                0, _D, dim_body, (zero, zero, zero))
            res = (_sqrt16(acc_dst)
                   + jnp.abs(_sqrt16(acc_c) - 1.0)
                   + jnp.abs(_sqrt16(acc_d) - 1.0))
            o_v[pl.ds(chunk * _K + g * _L, _L)] = res
            return 0

        lax.fori_loop(0, _K // _L, group_body, 0)

    pltpu.sync_copy(o_v, out_hbm.at[pl.ds(base, _PW)])


_mesh = plsc.VectorSubcoreMesh(core_axis_name="c", subcore_axis_name="s")

_tb = functools.partial(
    pl.kernel,
    out_type=jax.ShapeDtypeStruct((_B,), jnp.float32),
    mesh=_mesh,
    scratch_types=[
        pltpu.VMEM((_K,), jnp.int32),
        pltpu.VMEM((_K,), jnp.int32),
        pltpu.VMEM((_K,), jnp.int32),
        pltpu.VMEM((_K, 2 * _D), jnp.float32),
        pltpu.VMEM((_K, 2 * _D), jnp.float32),
        pltpu.VMEM((_K, 2 * _D), jnp.float32),
        pltpu.VMEM((_PW,), jnp.float32),
        pltpu.SemaphoreType.DMA,
    ],
    compiler_params=pltpu.CompilerParams(needs_layout_passes=False),
)(_body)


@jax.jit
def kernel(input, class_emb, rel_emb):
    idx = input.astype(jnp.int32)
    out = _tb(idx[:, 0], idx[:, 1], idx[:, 2], class_emb, rel_emb)
    return out.reshape(_B, 1)


# speedup vs baseline: 1.5358x; 1.0211x over previous
import functools

import jax
import jax.numpy as jnp
from jax import lax
from jax.experimental import pallas as pl
from jax.experimental.pallas import tpu as pltpu
from jax.experimental.pallas import tpu_sc as plsc

_D = 64
_B = 16384
_NC = 2
_NS = 16
_NW = _NC * _NS
_L = 16
_K = 128
_PW = _B // _NW
_NCHUNK = _PW // _K


def _sqrt16(x):
    i = plsc.bitcast(x, jnp.int32)
    y = plsc.bitcast((i >> 1) + jnp.int32(0x1FBD1DF5), jnp.float32)
    y = 0.5 * (y + x / y)
    y = 0.5 * (y + x / y)
    y = 0.5 * (y + x / y)
    return y


def _body(i0_hbm, i1_hbm, i2_hbm, cls_hbm, rel_hbm, out_hbm,
          i0_a, i1_a, i2_a, c_a, r_a, d_a,
          i0_b, i1_b, i2_b, c_b, r_b, d_b,
          o_v, sem_a, sem_b):
    wid = lax.axis_index("s") * _NC + lax.axis_index("c")
    base = wid * _PW

    lanes = lax.iota(jnp.int32, _L)
    bufs = [(i0_a, i1_a, i2_a, c_a, r_a, d_a, sem_a),
            (i0_b, i1_b, i2_b, c_b, r_b, d_b, sem_b)]

    def issue(buf, chunk):
        i0_v, i1_v, i2_v, c_v, r_v, d_v, sem = buf
        cb = base + chunk * _K
        pltpu.sync_copy(i0_hbm.at[pl.ds(cb, _K)], i0_v)
        pltpu.sync_copy(i1_hbm.at[pl.ds(cb, _K)], i1_v)
        pltpu.sync_copy(i2_hbm.at[pl.ds(cb, _K)], i2_v)
        return (pltpu.async_copy(cls_hbm.at[i0_v], c_v, sem),
                pltpu.async_copy(rel_hbm.at[i1_v], r_v, sem),
                pltpu.async_copy(cls_hbm.at[i2_v], d_v, sem))

    def compute(buf, chunk):
        _, _, _, c_v, r_v, d_v, _ = buf

        def group_body(g, _):
            rows = lanes + g * _L
            zero = jnp.zeros((_L,), jnp.float32)
            acc_dst = zero
            acc_c = zero
            acc_d = zero
            for j in range(_D):
                jlo = jnp.full((_L,), j, jnp.int32)
                jhi = jnp.full((_L,), j + _D, jnp.int32)
                c1 = plsc.load_gather(c_v, [rows, jlo])
                c2 = plsc.load_gather(c_v, [rows, jhi])
                d1 = plsc.load_gather(d_v, [rows, jlo])
                d2 = plsc.load_gather(d_v, [rows, jhi])
                r1 = plsc.load_gather(r_v, [rows, jlo])
                r2 = plsc.load_gather(r_v, [rows, jhi])
                t = (jnp.abs(c1 - d1 - r1) + jnp.abs(c2) + jnp.abs(d2)
                     - jnp.abs(r2))
                t = jnp.maximum(t, 0.0)
                acc_dst = acc_dst + t * t
                acc_c = acc_c + c1 * c1
                acc_d = acc_d + d1 * d1
            res = (_sqrt16(acc_dst)
                   + jnp.abs(_sqrt16(acc_c) - 1.0)
                   + jnp.abs(_sqrt16(acc_d) - 1.0))
            o_v[pl.ds(chunk * _K + g * _L, _L)] = res
            return 0

        lax.fori_loop(0, _K // _L, group_body, 0)

    copies = issue(bufs[0], 0)
    for chunk in range(_NCHUNK):
        if chunk + 1 < _NCHUNK:
            next_copies = issue(bufs[(chunk + 1) % 2], chunk + 1)
        for cp in copies:
            cp.wait()
        compute(bufs[chunk % 2], chunk)
        if chunk + 1 < _NCHUNK:
            copies = next_copies

    pltpu.sync_copy(o_v, out_hbm.at[pl.ds(base, _PW)])


_mesh = plsc.VectorSubcoreMesh(core_axis_name="c", subcore_axis_name="s")

_dbuf = [
    pltpu.VMEM((_K,), jnp.int32),
    pltpu.VMEM((_K,), jnp.int32),
    pltpu.VMEM((_K,), jnp.int32),
    pltpu.VMEM((_K, 2 * _D), jnp.float32),
    pltpu.VMEM((_K, 2 * _D), jnp.float32),
    pltpu.VMEM((_K, 2 * _D), jnp.float32),
]

_tb = functools.partial(
    pl.kernel,
    out_type=jax.ShapeDtypeStruct((_B,), jnp.float32),
    mesh=_mesh,
    scratch_types=_dbuf + _dbuf + [
        pltpu.VMEM((_PW,), jnp.float32),
        pltpu.SemaphoreType.DMA,
        pltpu.SemaphoreType.DMA,
    ],
    compiler_params=pltpu.CompilerParams(needs_layout_passes=False),
)(_body)


@jax.jit
def kernel(input, class_emb, rel_emb):
    idx = input.astype(jnp.int32)
    out = _tb(idx[:, 0], idx[:, 1], idx[:, 2], class_emb, rel_emb)
    return out.reshape(_B, 1)


# speedup vs baseline: 4.4045x; 2.8679x over previous
import functools

import jax
import jax.numpy as jnp
from jax import lax
from jax.experimental import pallas as pl
from jax.experimental.pallas import tpu as pltpu
from jax.experimental.pallas import tpu_sc as plsc

_D = 64
_B = 16384
_NC = 2
_NS = 16
_NW = _NC * _NS
_L = 16
_K = 128
_PW = _B // _NW
_NCHUNK = _PW // _K


def _sqrt16(x):
    i = plsc.bitcast(x, jnp.int32)
    y = plsc.bitcast((i >> 1) + jnp.int32(0x1FBD1DF5), jnp.float32)
    y = 0.5 * (y + x / y)
    y = 0.5 * (y + x / y)
    y = 0.5 * (y + x / y)
    return y


def _body(i0_hbm, i1_hbm, i2_hbm, cls_hbm, rel_hbm, out_hbm,
          i0_a, i1_a, i2_a, c_a, r_a, d_a,
          i0_b, i1_b, i2_b, c_b, r_b, d_b,
          o_v, sem_a, sem_b):
    wid = lax.axis_index("s") * _NC + lax.axis_index("c")
    base = wid * _PW

    lanes = lax.iota(jnp.int32, _L)
    bufs = [(i0_a, i1_a, i2_a, c_a, r_a, d_a, sem_a),
            (i0_b, i1_b, i2_b, c_b, r_b, d_b, sem_b)]

    def issue(buf, chunk):
        i0_v, i1_v, i2_v, c_v, r_v, d_v, sem = buf
        cb = base + chunk * _K
        pltpu.sync_copy(i0_hbm.at[pl.ds(cb, _K)], i0_v)
        pltpu.sync_copy(i1_hbm.at[pl.ds(cb, _K)], i1_v)
        pltpu.sync_copy(i2_hbm.at[pl.ds(cb, _K)], i2_v)
        return (pltpu.async_copy(cls_hbm.at[i0_v], c_v, sem),
                pltpu.async_copy(rel_hbm.at[i1_v], r_v, sem),
                pltpu.async_copy(cls_hbm.at[i2_v], d_v, sem))

    def compute(buf, chunk):
        _, _, _, c_v, r_v, d_v, _ = buf

        def group_body(g, _):
            rows = lanes + g * _L
            zero = jnp.zeros((_L,), jnp.float32)
            acc_dst = zero
            acc_c = zero
            acc_d = zero
            jlo = lanes
            for j in range(_D):
                jhi = jlo + _D
                c1 = plsc.load_gather(c_v, [rows, jlo])
                c2 = plsc.load_gather(c_v, [rows, jhi])
                d1 = plsc.load_gather(d_v, [rows, jlo])
                d2 = plsc.load_gather(d_v, [rows, jhi])
                r1 = plsc.load_gather(r_v, [rows, jlo])
                r2 = plsc.load_gather(r_v, [rows, jhi])
                t = (jnp.abs(c1 - d1 - r1) + jnp.abs(c2) + jnp.abs(d2)
                     - jnp.abs(r2))
                t = jnp.maximum(t, 0.0)
                acc_dst = acc_dst + t * t
                acc_c = acc_c + c1 * c1
                acc_d = acc_d + d1 * d1
                jlo = jlo + 1
                jlo = jnp.where(jlo == _D, 0, jlo)
            res = (_sqrt16(acc_dst)
                   + jnp.abs(_sqrt16(acc_c) - 1.0)
                   + jnp.abs(_sqrt16(acc_d) - 1.0))
            o_v[pl.ds(chunk * _K + g * _L, _L)] = res
            return 0

        lax.fori_loop(0, _K // _L, group_body, 0)

    copies = issue(bufs[0], 0)
    for chunk in range(_NCHUNK):
        if chunk + 1 < _NCHUNK:
            next_copies = issue(bufs[(chunk + 1) % 2], chunk + 1)
        for cp in copies:
            cp.wait()
        compute(bufs[chunk % 2], chunk)
        if chunk + 1 < _NCHUNK:
            copies = next_copies

    pltpu.sync_copy(o_v, out_hbm.at[pl.ds(base, _PW)])


_mesh = plsc.VectorSubcoreMesh(core_axis_name="c", subcore_axis_name="s")

_dbuf = [
    pltpu.VMEM((_K,), jnp.int32),
    pltpu.VMEM((_K,), jnp.int32),
    pltpu.VMEM((_K,), jnp.int32),
    pltpu.VMEM((_K, 2 * _D), jnp.float32),
    pltpu.VMEM((_K, 2 * _D), jnp.float32),
    pltpu.VMEM((_K, 2 * _D), jnp.float32),
]

_tb = functools.partial(
    pl.kernel,
    out_type=jax.ShapeDtypeStruct((_B,), jnp.float32),
    mesh=_mesh,
    scratch_types=_dbuf + _dbuf + [
        pltpu.VMEM((_PW,), jnp.float32),
        pltpu.SemaphoreType.DMA,
        pltpu.SemaphoreType.DMA,
    ],
    compiler_params=pltpu.CompilerParams(needs_layout_passes=False),
)(_body)


@jax.jit
def kernel(input, class_emb, rel_emb):
    idx = input.astype(jnp.int32)
    out = _tb(idx[:, 0], idx[:, 1], idx[:, 2], class_emb, rel_emb)
    return out.reshape(_B, 1)
